# baseline (speedup 1.0000x reference)
import functools

import jax
import jax.numpy as jnp
from jax import lax
from jax.experimental import pallas as pl
from jax.experimental.pallas import tpu as pltpu
from jax.experimental.pallas import tpu_sc as plsc

_N_STATES = 256
_N_OBVS = 100000
_BATCH = 16384

_NC = 2
_NS = 16
_NW = _NC * _NS
_LANES = 16

_HALF = _N_STATES // 2

_CHUNK = 128
_ROWS_PER_W = _BATCH // (_NW * _CHUNK)


def _emission_gather(top_flat, bot_flat, state2d, obs2d):
    mesh = plsc.VectorSubcoreMesh(core_axis_name="c", subcore_axis_name="s")

    @functools.partial(
        pl.kernel,
        mesh=mesh,
        out_type=jax.ShapeDtypeStruct((_BATCH // _CHUNK, _CHUNK), jnp.float32),
        scratch_types=[
            pltpu.VMEM((_ROWS_PER_W, _CHUNK), jnp.int32),
            pltpu.VMEM((_ROWS_PER_W, _CHUNK), jnp.int32),
            pltpu.VMEM((_ROWS_PER_W, _CHUNK), jnp.int32),
            pltpu.VMEM((_ROWS_PER_W, _CHUNK), jnp.int32),
            pltpu.VMEM((_ROWS_PER_W, _CHUNK), jnp.float32),
            pltpu.VMEM((_ROWS_PER_W, _CHUNK), jnp.float32),
            pltpu.VMEM((_ROWS_PER_W, _CHUNK), jnp.float32),
            pltpu.SemaphoreType.DMA,
        ],
    )
    def k(top_hbm, bot_hbm, state_hbm, obs_hbm, out_hbm,
          st_v, ob_v, it_v, ib_v, vt_v, vb_v, val_v, sem):
        wid = lax.axis_index("s") * _NC + lax.axis_index("c")
        base = wid * _ROWS_PER_W
        pltpu.sync_copy(state_hbm.at[pl.ds(base, _ROWS_PER_W)], st_v)
        pltpu.sync_copy(obs_hbm.at[pl.ds(base, _ROWS_PER_W)], ob_v)
        for j in range(_ROWS_PER_W):
            for t in range(_CHUNK // _LANES):
                sl = pl.ds(t * _LANES, _LANES)
                r = st_v[j, sl]
                c = ob_v[j, sl]
                it_v[j, sl] = jnp.minimum(r, _HALF - 1) * _N_OBVS + c
                ib_v[j, sl] = (jnp.maximum(r, _HALF) - _HALF) * _N_OBVS + c
        copies = []
        for j in range(_ROWS_PER_W):
            copies.append(
                pltpu.async_copy(top_hbm.at[it_v.at[j]], vt_v.at[j], sem))
            copies.append(
                pltpu.async_copy(bot_hbm.at[ib_v.at[j]], vb_v.at[j], sem))
        for c in copies:
            c.wait()
        for j in range(_ROWS_PER_W):
            for t in range(_CHUNK // _LANES):
                sl = pl.ds(t * _LANES, _LANES)
                val_v[j, sl] = jnp.where(
                    st_v[j, sl] < _HALF, vt_v[j, sl], vb_v[j, sl])
        pltpu.sync_copy(val_v, out_hbm.at[pl.ds(base, _ROWS_PER_W)])

    return k(top_flat, bot_flat, state2d, obs2d)


def kernel(log_em, state, obs):
    top_flat = log_em[:_HALF].reshape(-1)
    bot_flat = log_em[_HALF:].reshape(-1)
    state2d = state.reshape(_BATCH // _CHUNK, _CHUNK)
    obs2d = obs.reshape(_BATCH // _CHUNK, _CHUNK)
    out2d = _emission_gather(top_flat, bot_flat, state2d, obs2d)
    return out2d.reshape(-1)


# speedup vs baseline: 1.3415x; 1.3415x over previous
import functools

import jax
import jax.numpy as jnp
from jax import lax
from jax.experimental import pallas as pl
from jax.experimental.pallas import tpu as pltpu
from jax.experimental.pallas import tpu_sc as plsc

_N_STATES = 256
_N_OBVS = 100000
_BATCH = 16384

_NC = 2
_NS = 16
_NW = _NC * _NS
_LANES = 16

_CHUNK = 128
_ROWS_PER_W = _BATCH // (_NW * _CHUNK)


def _emission_gather(table, state2d, obs2d):
    mesh = plsc.VectorSubcoreMesh(core_axis_name="c", subcore_axis_name="s")

    @functools.partial(
        pl.kernel,
        mesh=mesh,
        compiler_params=pltpu.CompilerParams(use_tc_tiling_on_sc=False),
        out_type=jax.ShapeDtypeStruct((_BATCH // _CHUNK, _CHUNK), jnp.float32),
        scratch_types=[
            pltpu.VMEM((_ROWS_PER_W, _CHUNK), jnp.int32),
            pltpu.VMEM((_ROWS_PER_W, _CHUNK), jnp.int32),
            pltpu.VMEM((_ROWS_PER_W, _CHUNK), jnp.int32),
            pltpu.VMEM((_ROWS_PER_W, _CHUNK), jnp.float32),
            pltpu.SemaphoreType.DMA,
        ],
    )
    def k(table_hbm, state_hbm, obs_hbm, out_hbm, st_v, ob_v, idx_v, val_v, sem):
        wid = lax.axis_index("s") * _NC + lax.axis_index("c")
        base = wid * _ROWS_PER_W
        pltpu.sync_copy(state_hbm.at[pl.ds(base, _ROWS_PER_W)], st_v)
        pltpu.sync_copy(obs_hbm.at[pl.ds(base, _ROWS_PER_W)], ob_v)
        for j in range(_ROWS_PER_W):
            for t in range(_CHUNK // _LANES):
                sl = pl.ds(t * _LANES, _LANES)
                idx_v[j, sl] = st_v[j, sl] * _N_OBVS + ob_v[j, sl]
        row0 = table_hbm.at[0]
        copies = [
            pltpu.async_copy(row0.at[idx_v.at[j]], val_v.at[j], sem)
            for j in range(_ROWS_PER_W)
        ]
        for c in copies:
            c.wait()
        pltpu.sync_copy(val_v, out_hbm.at[pl.ds(base, _ROWS_PER_W)])

    return k(table, state2d, obs2d)


def kernel(log_em, state, obs):
    state2d = state.reshape(_BATCH // _CHUNK, _CHUNK)
    obs2d = obs.reshape(_BATCH // _CHUNK, _CHUNK)
    out2d = _emission_gather(log_em, state2d, obs2d)
    return out2d.reshape(-1)


# speedup vs baseline: 1.3440x; 1.0019x over previous
import functools

import jax
import jax.numpy as jnp
from jax import lax
from jax.experimental import pallas as pl
from jax.experimental.pallas import tpu as pltpu
from jax.experimental.pallas import tpu_sc as plsc

_N_STATES = 256
_N_OBVS = 100000
_BATCH = 16384

_NC = 2
_NS = 16
_NW = _NC * _NS
_LANES = 16

_CHUNK = 128
_ROWS_PER_W = _BATCH // (_NW * _CHUNK)


def _emission_gather(table, state2d, obs2d):
    mesh = plsc.VectorSubcoreMesh(core_axis_name="c", subcore_axis_name="s")

    @functools.partial(
        pl.kernel,
        mesh=mesh,
        compiler_params=pltpu.CompilerParams(use_tc_tiling_on_sc=False),
        out_type=jax.ShapeDtypeStruct((_BATCH // _CHUNK, _CHUNK), jnp.float32),
        scratch_types=[
            pltpu.VMEM((_ROWS_PER_W, _CHUNK), jnp.int32),
            pltpu.VMEM((_ROWS_PER_W, _CHUNK), jnp.int32),
            pltpu.VMEM((_ROWS_PER_W, _CHUNK), jnp.int32),
            pltpu.VMEM((_ROWS_PER_W, _CHUNK), jnp.float32),
            pltpu.SemaphoreType.DMA,
        ],
    )
    def k(table_hbm, state_hbm, obs_hbm, out_hbm, st_v, ob_v, idx_v, val_v, sem):
        wid = lax.axis_index("s") * _NC + lax.axis_index("c")
        base = wid * _ROWS_PER_W
        pltpu.sync_copy(state_hbm.at[pl.ds(base, _ROWS_PER_W)], st_v)
        pltpu.sync_copy(obs_hbm.at[pl.ds(base, _ROWS_PER_W)], ob_v)
        for j in range(_ROWS_PER_W):
            for t in range(_CHUNK // _LANES):
                sl = pl.ds(t * _LANES, _LANES)
                idx_v[j, sl] = st_v[j, sl] * _N_OBVS + ob_v[j, sl]
        copies = [
            pltpu.async_copy(table_hbm.at[idx_v.at[j]], val_v.at[j], sem)
            for j in range(_ROWS_PER_W)
        ]
        for c in copies:
            c.wait()
        pltpu.sync_copy(val_v, out_hbm.at[pl.ds(base, _ROWS_PER_W)])

    return k(table, state2d, obs2d)


def kernel(log_em, state, obs):
    table_flat = log_em.reshape(-1)
    state2d = state.reshape(_BATCH // _CHUNK, _CHUNK)
    obs2d = obs.reshape(_BATCH // _CHUNK, _CHUNK)
    out2d = _emission_gather(table_flat, state2d, obs2d)
    return out2d.reshape(-1)
